# baseline (speedup 1.0000x reference)
import functools

import jax
import jax.numpy as jnp
from jax import lax
from jax.experimental import pallas as pl
from jax.experimental.pallas import tpu as pltpu
from jax.experimental.pallas import tpu_sc as plsc

BATCH = 4096
MAX_LEN = 200
D_MODEL = 64
LANES = 16
NUM_CORES = 2
NUM_SUBCORES = 16
NUM_WORKERS = NUM_CORES * NUM_SUBCORES
BPW = BATCH // NUM_WORKERS
IPW = BPW * MAX_LEN
SPLIT = (104, 96)


def kernel(x, token_table, pos_table):
    x_flat = x.astype(jnp.int32).reshape(-1)
    pos_flat = pos_table.reshape(-1)
    mesh = plsc.VectorSubcoreMesh(core_axis_name="c", subcore_axis_name="s")

    @functools.partial(
        pl.kernel,
        out_type=jax.ShapeDtypeStruct((BATCH * MAX_LEN, D_MODEL), jnp.float32),
        mesh=mesh,
        compiler_params=pltpu.CompilerParams(use_tc_tiling_on_sc=False),
        scratch_types=[
            pltpu.VMEM((IPW,), jnp.int32),
            pltpu.VMEM((MAX_LEN * D_MODEL,), jnp.float32),
            pltpu.VMEM((SPLIT[0], D_MODEL), jnp.float32),
            pltpu.VMEM((SPLIT[1], D_MODEL), jnp.float32),
            pltpu.VMEM((SPLIT[0], D_MODEL), jnp.float32),
            pltpu.VMEM((SPLIT[1], D_MODEL), jnp.float32),
        ] + [pltpu.SemaphoreType.DMA] * 8,
    )
    def sc_kernel(x_hbm, tok_hbm, pos_hbm, out_hbm, idx_v, pos_v,
                  r00, r01, r10, r11, *sems):
        rows = ((r00, r01), (r10, r11))
        gsem = (sems[0:2], sems[2:4])
        osem = (sems[4:6], sems[6:8])
        wid = lax.axis_index("s") * NUM_CORES + lax.axis_index("c")
        i0 = wid * IPW
        pltpu.sync_copy(x_hbm.at[pl.ds(i0, IPW)], idx_v)
        pltpu.sync_copy(pos_hbm, pos_v)

        def gather_op(r, a, c):
            l0 = 0 if c == 0 else SPLIT[0]
            return pltpu.make_async_copy(
                tok_hbm.at[idx_v.at[pl.ds(r * MAX_LEN + l0, SPLIT[c])]],
                rows[a][c], gsem[a][c])

        def put_op(r, a, c):
            l0 = 0 if c == 0 else SPLIT[0]
            return pltpu.make_async_copy(
                rows[a][c],
                out_hbm.at[pl.ds(i0 + r * MAX_LEN + l0, SPLIT[c])],
                osem[a][c])

        for a in range(2):
            for c in range(2):
                gather_op(a, a, c).start()

        @pl.loop(0, BPW, step=2)
        def _(g):
            for a in range(2):
                for c in range(2):
                    r = g + a
                    l0 = 0 if c == 0 else SPLIT[0]
                    gather_op(r, a, c).wait()
                    buf = rows[a][c]

                    @pl.loop(0, SPLIT[c])
                    def _(i):
                        for j in range(D_MODEL // LANES):
                            sl = pl.ds(j * LANES, LANES)
                            plsc.addupdate(
                                buf.at[i, sl],
                                pos_v[pl.ds((l0 + i) * D_MODEL + j * LANES,
                                            LANES)])

                    put_op(r, a, c).start()
            for a in range(2):
                for c in range(2):
                    r = g + a
                    put_op(r, a, c).wait()

                    @pl.when(r + 2 < BPW)
                    def _():
                        gather_op(r + 2, a, c).start()

    flat = sc_kernel(x_flat, token_table, pos_flat)
    return flat.reshape(BATCH, MAX_LEN, D_MODEL)


# speedup vs baseline: 1.3087x; 1.3087x over previous
import functools

import jax
import jax.numpy as jnp
from jax import lax
from jax.experimental import pallas as pl
from jax.experimental.pallas import tpu as pltpu
from jax.experimental.pallas import tpu_sc as plsc

BATCH = 4096
MAX_LEN = 200
D_MODEL = 64
LANES = 16
NUM_CORES = 2
NUM_SUBCORES = 16
NUM_WORKERS = NUM_CORES * NUM_SUBCORES
BPW = BATCH // NUM_WORKERS
IPW = BPW * MAX_LEN
SPLIT = (104, 96)


def kernel(x, token_table, pos_table):
    x_flat = x.astype(jnp.int32).reshape(-1)
    pos_flat = pos_table.reshape(-1)
    mesh = plsc.VectorSubcoreMesh(core_axis_name="c", subcore_axis_name="s")

    @functools.partial(
        pl.kernel,
        out_type=jax.ShapeDtypeStruct((BATCH * MAX_LEN, 2 * D_MODEL),
                                      jnp.float32),
        mesh=mesh,
        compiler_params=pltpu.CompilerParams(use_tc_tiling_on_sc=False),
        scratch_types=[
            pltpu.VMEM((IPW,), jnp.int32),
            pltpu.VMEM((MAX_LEN * D_MODEL,), jnp.float32),
            pltpu.VMEM((SPLIT[0], D_MODEL), jnp.float32),
            pltpu.VMEM((SPLIT[1], D_MODEL), jnp.float32),
            pltpu.VMEM((SPLIT[0], D_MODEL), jnp.float32),
            pltpu.VMEM((SPLIT[1], D_MODEL), jnp.float32),
        ] + [pltpu.SemaphoreType.DMA] * 8,
    )
    def sc_kernel(x_hbm, tok_hbm, pos_hbm, out_hbm, idx_v, pos_v,
                  r00, r01, r10, r11, *sems):
        rows = ((r00, r01), (r10, r11))
        gsem = (sems[0:2], sems[2:4])
        osem = (sems[4:6], sems[6:8])
        wid = lax.axis_index("s") * NUM_CORES + lax.axis_index("c")
        i0 = wid * IPW
        pltpu.sync_copy(x_hbm.at[pl.ds(i0, IPW)], idx_v)
        pltpu.sync_copy(pos_hbm, pos_v)

        def gather_op(r, a, c):
            l0 = 0 if c == 0 else SPLIT[0]
            return pltpu.make_async_copy(
                tok_hbm.at[idx_v.at[pl.ds(r * MAX_LEN + l0, SPLIT[c])]],
                rows[a][c], gsem[a][c])

        def put_op(r, a, c):
            l0 = 0 if c == 0 else SPLIT[0]
            return pltpu.make_async_copy(
                rows[a][c],
                out_hbm.at[pl.ds(i0 + r * MAX_LEN + l0, SPLIT[c]),
                           pl.ds(0, D_MODEL)],
                osem[a][c])

        for a in range(2):
            for c in range(2):
                gather_op(a, a, c).start()

        @pl.loop(0, BPW, step=2)
        def _(g):
            for a in range(2):
                for c in range(2):
                    r = g + a
                    l0 = 0 if c == 0 else SPLIT[0]
                    gather_op(r, a, c).wait()
                    buf = rows[a][c]

                    @pl.loop(0, SPLIT[c])
                    def _(i):
                        for j in range(D_MODEL // LANES):
                            sl = pl.ds(j * LANES, LANES)
                            plsc.addupdate(
                                buf.at[i, sl],
                                pos_v[pl.ds((l0 + i) * D_MODEL + j * LANES,
                                            LANES)])

                    put_op(r, a, c).start()
            for a in range(2):
                for c in range(2):
                    r = g + a
                    put_op(r, a, c).wait()

                    @pl.when(r + 2 < BPW)
                    def _():
                        gather_op(r + 2, a, c).start()

    wide = sc_kernel(x_flat, token_table, pos_flat)
    return wide[:, :D_MODEL].reshape(BATCH, MAX_LEN, D_MODEL)


# speedup vs baseline: 1.4098x; 1.0773x over previous
import functools

import jax
import jax.numpy as jnp
from jax import lax
from jax.experimental import pallas as pl
from jax.experimental.pallas import tpu as pltpu
from jax.experimental.pallas import tpu_sc as plsc

BATCH = 4096
MAX_LEN = 200
D_MODEL = 64
LANES = 16
NUM_CORES = 2
NUM_SUBCORES = 16
NUM_WORKERS = NUM_CORES * NUM_SUBCORES
BPW = BATCH // NUM_WORKERS
IPW = BPW * MAX_LEN
ROW_BYTES = MAX_LEN * D_MODEL * 4


def kernel(x, token_table, pos_table):
    mesh = plsc.VectorSubcoreMesh(core_axis_name="c", subcore_axis_name="s")

    @functools.partial(
        pl.kernel,
        out_type=jax.ShapeDtypeStruct((BATCH, MAX_LEN, D_MODEL), jnp.float32),
        mesh=mesh,
        compiler_params=pltpu.CompilerParams(use_tc_tiling_on_sc=True),
        scratch_types=[
            pltpu.VMEM((4, MAX_LEN), jnp.int32),
            pltpu.VMEM((MAX_LEN, D_MODEL), jnp.float32),
            pltpu.VMEM((MAX_LEN, D_MODEL), jnp.float32),
            pltpu.VMEM((MAX_LEN, D_MODEL), jnp.float32),
            pltpu.VMEM((MAX_LEN, D_MODEL), jnp.float32),
            pltpu.VMEM((MAX_LEN, D_MODEL), jnp.float32),
        ] + [pltpu.SemaphoreType.DMA] * 12,
    )
    def sc_kernel(x_hbm, tok_hbm, pos_hbm, out_hbm, idx_v, pos_v,
                  b0_, b1_, b2_, b3_, *sems):
        bufs = (b0_, b1_, b2_, b3_)
        gsem = sems[0:4]
        osem = sems[4:8]
        isem = sems[8:12]
        wid = lax.axis_index("s") * NUM_CORES + lax.axis_index("c")
        row0 = wid * BPW
        pltpu.sync_copy(pos_hbm, pos_v)

        def idx_op(b, s):
            return pltpu.make_async_copy(
                x_hbm.at[row0 + b], idx_v.at[s], isem[s])

        def fire_gathers(q):
            @pl.loop(0, MAX_LEN - 8, step=LANES)
            def _(i):
                v = idx_v[q, pl.ds(i, LANES)]
                for u in range(LANES):
                    pltpu.make_async_copy(
                        tok_hbm.at[v[u]], bufs[q].at[i + u], gsem[q]).start()

            vt = idx_v[q, pl.ds(MAX_LEN - LANES, LANES)]
            for u in range(8):
                pltpu.make_async_copy(
                    tok_hbm.at[vt[8 + u]], bufs[q].at[MAX_LEN - 8 + u],
                    gsem[q]).start()

        def wait_gathers(q):
            pltpu.make_async_copy(
                tok_hbm.at[pl.ds(0, MAX_LEN)], bufs[q], gsem[q]).wait()

        def fire_out(b, q):
            pltpu.make_async_copy(
                bufs[q], out_hbm.at[row0 + b], osem[q]).start()

        def wait_out(b, q):
            pltpu.make_async_copy(
                bufs[q], out_hbm.at[row0 + b], osem[q]).wait()

        for s in range(4):
            idx_op(s, s).start()
        for q in range(2):
            idx_op(q, q).wait()
            fire_gathers(q)

        @pl.loop(0, BPW, step=4)
        def _(g):
            for q in range(4):
                b = g + q
                nf = b + 4

                @pl.when(nf < BPW)
                def _():
                    idx_op(nf, q).start()

                wait_gathers(q)

                @pl.loop(0, MAX_LEN)
                def _(i):
                    for j in range(D_MODEL // LANES):
                        sl = pl.ds(j * LANES, LANES)
                        plsc.addupdate(bufs[q].at[i, sl], pos_v[i, sl])

                fire_out(b, q)
                nxt = b + 2
                qn = (q + 2) % 4

                @pl.when(nxt < BPW)
                def _():
                    @pl.when(nxt >= 4)
                    def _():
                        wait_out(nxt - 4, qn)

                    idx_op(nxt, qn).wait()
                    fire_gathers(qn)

        for b in range(BPW - 4, BPW):
            wait_out(b, b % 4)

    return sc_kernel(x.astype(jnp.int32), token_table, pos_table)
